# baseline (speedup 1.0000x reference)
import math
import functools

import jax
import jax.numpy as jnp
from jax.experimental import pallas as pl
from jax.experimental.pallas import tpu as pltpu

B = 1
T = 2048
D_MODEL = 768
H = 12
DH = D_MODEL // H
P = 8
S = B * T * H

_INV_SQRT_DH = 1.0 / math.sqrt(DH)


def _fused_body(x_ref, w1c_ref, w2_ref, f_ref, e_ref,
                pq_ref, gq_ref, pk_ref, gk_ref, pv_ref, gv_ref, wot_ref,
                out_ref, logq_ref, logk_ref, logv_ref,
                rawq_ref, rawk_ref, rawv_ref,
                kall, vall, *, bt):
    i = pl.program_id(0)
    h = pl.program_id(1)
    xh = x_ref[0]
    fmat = f_ref[...]
    emat = e_ref[...]

    h_all = jnp.maximum(
        jax.lax.dot_general(xh, w1c_ref[...], (((1,), (0,)), ((), ())),
                            preferred_element_type=jnp.float32), 0.0)
    eo_parts = [
        jax.lax.dot_general(h_all[:, p * DH:(p + 1) * DH], w2_ref[p],
                            (((1,), (0,)), ((), ())),
                            preferred_element_type=jnp.float32)
        for p in range(P)
    ]
    eo_all = jnp.concatenate(eo_parts, axis=1)
    norm = jnp.sqrt(jax.lax.dot_general(
        eo_all * eo_all, fmat, (((1,), (0,)), ((), ())),
        preferred_element_type=jnp.float32))

    def gate(p_ref, g_ref, log_ref, raw_ref):
        raw = jax.lax.dot_general(xh, p_ref[...], (((1,), (1,)), ((), ())),
                                  preferred_element_type=jnp.float32)
        raw = raw * _INV_SQRT_DH - g_ref[...]
        logit = jnp.maximum(raw, 0.0)
        w = jnp.where(logit > 1e-6, logit, 0.0)
        wrep = jax.lax.dot_general(w, fmat, (((1,), (1,)), ((), ())),
                                   preferred_element_type=jnp.float32)
        syn = jax.lax.dot_general(eo_all * wrep, emat,
                                  (((1,), (0,)), ((), ())),
                                  preferred_element_type=jnp.float32)
        log_ref[:, pl.ds(h, 1), :] = logit[:, None, :]
        raw_ref[:, pl.ds(h, 1), :] = (w * norm)[:, None, :]
        return syn

    synq = gate(pq_ref, gq_ref, logq_ref, rawq_ref)
    synk = gate(pk_ref, gk_ref, logk_ref, rawk_ref)
    synv = gate(pv_ref, gv_ref, logv_ref, rawv_ref)

    kall[pl.ds(h, 1), pl.ds(i * bt, bt), :] = synk[None]
    vall[pl.ds(h, 1), pl.ds(i * bt, bt), :] = synv[None]

    def body(j, carry):
        acc, m, l = carry
        kb = kall[h, pl.ds(j * bt, bt), :]
        vb = vall[h, pl.ds(j * bt, bt), :]
        s = jax.lax.dot_general(synq, kb, (((1,), (1,)), ((), ())),
                                preferred_element_type=jnp.float32)
        s = s * _INV_SQRT_DH
        m_new = jnp.maximum(m, jnp.max(s, axis=1, keepdims=True))
        alpha = jnp.exp(m - m_new)
        pmat = jnp.exp(s - m_new)
        l = l * alpha + jnp.sum(pmat, axis=1, keepdims=True)
        acc = acc * alpha + jax.lax.dot_general(
            pmat, vb, (((1,), (0,)), ((), ())),
            preferred_element_type=jnp.float32)
        return acc, m_new, l

    acc0 = jnp.zeros((bt, DH), jnp.float32)
    m0 = jnp.full((bt, 1), -jnp.inf, jnp.float32)
    l0 = jnp.zeros((bt, 1), jnp.float32)
    acc, m, l = jax.lax.fori_loop(0, i, body, (acc0, m0, l0))

    s = jax.lax.dot_general(synq, synk, (((1,), (1,)), ((), ())),
                            preferred_element_type=jnp.float32)
    s = s * _INV_SQRT_DH
    rows = jax.lax.broadcasted_iota(jnp.int32, (bt, bt), 0)
    cols = jax.lax.broadcasted_iota(jnp.int32, (bt, bt), 1)
    s = jnp.where(rows >= cols, s, -jnp.inf)
    m_new = jnp.maximum(m, jnp.max(s, axis=1, keepdims=True))
    alpha = jnp.exp(m - m_new)
    pmat = jnp.exp(s - m_new)
    l = l * alpha + jnp.sum(pmat, axis=1, keepdims=True)
    acc = acc * alpha + jax.lax.dot_general(
        pmat, synv, (((1,), (0,)), ((), ())),
        preferred_element_type=jnp.float32)
    attn = acc / l

    contrib = jax.lax.dot_general(attn, wot_ref[...], (((1,), (0,)), ((), ())),
                                  preferred_element_type=jnp.float32)

    @pl.when(h == 0)
    def _():
        out_ref[...] = contrib

    @pl.when(h != 0)
    def _():
        out_ref[...] += contrib


def _fused(xhm, w1cat, W2, fmat, emat,
           proto_q, gate_q, proto_k, gate_k, proto_v, gate_v, WoT, bt=256):
    grid = (T // bt, H)
    full = lambda shape: pl.BlockSpec(shape, lambda i, h: tuple(0 for _ in shape))
    logspec = pl.BlockSpec((bt, H, P), lambda i, h: (i, 0, 0))
    out_shapes = (
        [jax.ShapeDtypeStruct((T, D_MODEL), jnp.float32)]
        + [jax.ShapeDtypeStruct((T, H, P), jnp.float32)] * 6
    )
    return pl.pallas_call(
        functools.partial(_fused_body, bt=bt),
        grid=grid,
        in_specs=[pl.BlockSpec((1, bt, DH), lambda i, h: (h, i, 0)),
                  full((DH, P * DH)), full((P, DH, DH)),
                  full((P * DH, P)), full((P * DH, DH)),
                  full((P, DH)), full((1, P)),
                  full((P, DH)), full((1, P)),
                  full((P, DH)), full((1, P)),
                  pl.BlockSpec((DH, D_MODEL), lambda i, h: (h, 0))],
        out_specs=[pl.BlockSpec((bt, D_MODEL), lambda i, h: (i, 0)),
                   logspec, logspec, logspec, logspec, logspec, logspec],
        out_shape=out_shapes,
        scratch_shapes=[pltpu.VMEM((H, T, DH), jnp.float32),
                        pltpu.VMEM((H, T, DH), jnp.float32)],
    )(xhm, w1cat, W2, fmat, emat,
      proto_q, gate_q, proto_k, gate_k, proto_v, gate_v, WoT)


def kernel(x, position_ids, proto_q, gate_q, proto_k, gate_k, proto_v, gate_v,
           W1, W2, Wo):
    xhm = x.reshape(T, H, DH).transpose(1, 0, 2)
    w1cat = jnp.transpose(W1, (1, 0, 2)).reshape(DH, P * DH)
    WoT = Wo.T
    ridx = jnp.arange(P * DH, dtype=jnp.int32)
    fmat = (ridx[:, None] // DH == jnp.arange(P, dtype=jnp.int32)[None, :]
            ).astype(jnp.float32)
    emat = (ridx[:, None] % DH == jnp.arange(DH, dtype=jnp.int32)[None, :]
            ).astype(jnp.float32)

    (out, logq, logk, logv, rawq, rawk, rawv) = _fused(
        xhm, w1cat, W2, fmat, emat,
        proto_q, gate_q.reshape(1, P),
        proto_k, gate_k.reshape(1, P), proto_v, gate_v.reshape(1, P), WoT)

    shape_log = (B, T, H, P)
    return (out.reshape(B, T, D_MODEL),
            logq.reshape(shape_log), logk.reshape(shape_log),
            logv.reshape(shape_log),
            rawq.reshape(S, P), rawk.reshape(S, P), rawv.reshape(S, P))


# speedup vs baseline: 1.1865x; 1.1865x over previous
import math
import functools

import jax
import jax.numpy as jnp
from jax.experimental import pallas as pl

B = 1
T = 2048
D_MODEL = 768
H = 12
DH = D_MODEL // H
P = 8
S = B * T * H

_INV_SQRT_DH = 1.0 / math.sqrt(DH)


def _compose_body(x_ref, w1c_ref, w2_ref, f_ref, e_ref,
                  pq_ref, gq_ref, pk_ref, gk_ref, pv_ref, gv_ref,
                  synq_ref, synk_ref, synv_ref, sml_ref):
    xb = x_ref[...]
    fmat = f_ref[...]
    emat = e_ref[...]

    h_all = jnp.maximum(
        jax.lax.dot_general(xb, w1c_ref[...], (((1,), (0,)), ((), ())),
                            preferred_element_type=jnp.float32), 0.0)
    eo_parts = [
        jax.lax.dot_general(h_all[:, p * DH:(p + 1) * DH], w2_ref[p],
                            (((1,), (0,)), ((), ())),
                            preferred_element_type=jnp.float32)
        for p in range(P)
    ]
    eo_all = jnp.concatenate(eo_parts, axis=1)
    norm = jnp.sqrt(jax.lax.dot_general(
        eo_all * eo_all, fmat, (((1,), (0,)), ((), ())),
        preferred_element_type=jnp.float32))

    def one(p_ref, g_ref, syn_ref):
        raw = jax.lax.dot_general(xb, p_ref[...], (((1,), (1,)), ((), ())),
                                  preferred_element_type=jnp.float32)
        raw = raw * _INV_SQRT_DH - g_ref[...]
        logit = jnp.maximum(raw, 0.0)
        w = jnp.where(logit > 1e-6, logit, 0.0)
        wrep = jax.lax.dot_general(w, fmat, (((1,), (1,)), ((), ())),
                                   preferred_element_type=jnp.float32)
        syn_ref[...] = jax.lax.dot_general(
            eo_all * wrep, emat, (((1,), (0,)), ((), ())),
            preferred_element_type=jnp.float32)
        return logit, w * norm

    logq, rawq = one(pq_ref, gq_ref, synq_ref)
    logk, rawk = one(pk_ref, gk_ref, synk_ref)
    logv, rawv = one(pv_ref, gv_ref, synv_ref)
    sml_ref[...] = jnp.concatenate([logq, logk, logv, rawq, rawk, rawv],
                                   axis=1)


def _compose(xf, w1cat, W2, fmat, emat,
             proto_q, gate_q, proto_k, gate_k, proto_v, gate_v, bs=512):
    grid = (S // bs,)
    row = pl.BlockSpec((bs, DH), lambda i: (i, 0))
    small = pl.BlockSpec((bs, 6 * P), lambda i: (i, 0))
    full = lambda shape: pl.BlockSpec(shape, lambda i: tuple(0 for _ in shape))
    out_shapes = (
        [jax.ShapeDtypeStruct((S, DH), jnp.float32)] * 3
        + [jax.ShapeDtypeStruct((S, 6 * P), jnp.float32)]
    )
    return pl.pallas_call(
        _compose_body,
        grid=grid,
        in_specs=[row, full((DH, P * DH)), full((P, DH, DH)),
                  full((P * DH, P)), full((P * DH, DH)),
                  full((P, DH)), full((1, P)),
                  full((P, DH)), full((1, P)),
                  full((P, DH)), full((1, P))],
        out_specs=[row, row, row, small],
        out_shape=out_shapes,
    )(xf, w1cat, W2, fmat, emat,
      proto_q, gate_q, proto_k, gate_k, proto_v, gate_v)


def _attn_body(q_ref, k_ref, v_ref, o_ref, *, bq, bk):
    i = pl.program_id(1)
    q = q_ref[0]

    def body(j, carry):
        acc, m, l = carry
        kb = k_ref[0, pl.ds(j * bk, bk), :]
        vb = v_ref[0, pl.ds(j * bk, bk), :]
        s = jax.lax.dot_general(q, kb, (((1,), (1,)), ((), ())),
                                preferred_element_type=jnp.float32)
        s = s * _INV_SQRT_DH
        m_new = jnp.maximum(m, jnp.max(s, axis=1, keepdims=True))
        alpha = jnp.exp(m - m_new)
        pmat = jnp.exp(s - m_new)
        l = l * alpha + jnp.sum(pmat, axis=1, keepdims=True)
        acc = acc * alpha + jax.lax.dot_general(
            pmat, vb, (((1,), (0,)), ((), ())),
            preferred_element_type=jnp.float32)
        return acc, m_new, l

    nfull = (i * bq) // bk
    acc = jnp.zeros((bq, DH), jnp.float32)
    m0 = jnp.full((bq, 1), -jnp.inf, jnp.float32)
    l0 = jnp.zeros((bq, 1), jnp.float32)
    acc, m, l = jax.lax.fori_loop(0, nfull, body, (acc, m0, l0))

    kb = k_ref[0, pl.ds(i * bq, bq), :]
    vb = v_ref[0, pl.ds(i * bq, bq), :]
    s = jax.lax.dot_general(q, kb, (((1,), (1,)), ((), ())),
                            preferred_element_type=jnp.float32)
    s = s * _INV_SQRT_DH
    rows = jax.lax.broadcasted_iota(jnp.int32, (bq, bq), 0)
    cols = jax.lax.broadcasted_iota(jnp.int32, (bq, bq), 1)
    s = jnp.where(rows >= cols, s, -jnp.inf)
    m_new = jnp.maximum(m, jnp.max(s, axis=1, keepdims=True))
    alpha = jnp.exp(m - m_new)
    pmat = jnp.exp(s - m_new)
    l = l * alpha + jnp.sum(pmat, axis=1, keepdims=True)
    acc = acc * alpha + jax.lax.dot_general(
        pmat, vb, (((1,), (0,)), ((), ())),
        preferred_element_type=jnp.float32)
    o_ref[0] = acc / l


def _attention(q, k, v, bq=512, bk=512):
    grid = (H, T // bq)
    qspec = pl.BlockSpec((1, bq, DH), lambda h, i: (h, i, 0))
    kvspec = pl.BlockSpec((1, T, DH), lambda h, i: (h, 0, 0))
    return pl.pallas_call(
        functools.partial(_attn_body, bq=bq, bk=bk),
        grid=grid,
        in_specs=[qspec, kvspec, kvspec],
        out_specs=qspec,
        out_shape=jax.ShapeDtypeStruct((H, T, DH), jnp.float32),
    )(q, k, v)


def _proj_body(x_ref, w_ref, o_ref):
    o_ref[...] = jax.lax.dot_general(x_ref[...], w_ref[...],
                                     (((1,), (1,)), ((), ())),
                                     preferred_element_type=jnp.float32)


def _out_proj(attn, Wo, br=512):
    grid = (T // br,)
    return pl.pallas_call(
        _proj_body,
        grid=grid,
        in_specs=[pl.BlockSpec((br, D_MODEL), lambda i: (i, 0)),
                  pl.BlockSpec((D_MODEL, D_MODEL), lambda i: (0, 0))],
        out_specs=pl.BlockSpec((br, D_MODEL), lambda i: (i, 0)),
        out_shape=jax.ShapeDtypeStruct((T, D_MODEL), jnp.float32),
    )(attn, Wo)


def kernel(x, position_ids, proto_q, gate_q, proto_k, gate_k, proto_v, gate_v,
           W1, W2, Wo):
    xf = x.reshape(S, DH)
    w1cat = jnp.transpose(W1, (1, 0, 2)).reshape(DH, P * DH)
    ridx = jnp.arange(P * DH, dtype=jnp.int32)
    fmat = (ridx[:, None] // DH == jnp.arange(P, dtype=jnp.int32)[None, :]
            ).astype(jnp.float32)
    emat = (ridx[:, None] % DH == jnp.arange(DH, dtype=jnp.int32)[None, :]
            ).astype(jnp.float32)

    (synq, synk, synv, sml) = _compose(
        xf, w1cat, W2, fmat, emat,
        proto_q, gate_q.reshape(1, P),
        proto_k, gate_k.reshape(1, P), proto_v, gate_v.reshape(1, P))

    q = synq.reshape(T, H, DH).transpose(1, 0, 2)
    k = synk.reshape(T, H, DH).transpose(1, 0, 2)
    v = synv.reshape(T, H, DH).transpose(1, 0, 2)
    attn = _attention(q, k, v).transpose(1, 0, 2).reshape(T, D_MODEL)
    out = _out_proj(attn, Wo).reshape(B, T, D_MODEL)

    shape_log = (B, T, H, P)
    return (out,
            sml[:, 0:P].reshape(shape_log), sml[:, P:2 * P].reshape(shape_log),
            sml[:, 2 * P:3 * P].reshape(shape_log),
            sml[:, 3 * P:4 * P], sml[:, 4 * P:5 * P], sml[:, 5 * P:6 * P])


# speedup vs baseline: 1.1980x; 1.0098x over previous
import math
import functools

import jax
import jax.numpy as jnp
from jax.experimental import pallas as pl

B = 1
T = 2048
D_MODEL = 768
H = 12
DH = D_MODEL // H
P = 8
S = B * T * H

_INV_SQRT_DH = 1.0 / math.sqrt(DH)


def _compose_body(x_ref, w1c_ref, w2_ref, f_ref, e_ref,
                  pq_ref, gq_ref, pk_ref, gk_ref, pv_ref, gv_ref,
                  synq_ref, synk_ref, synv_ref, sml_ref):
    xb = x_ref[...]
    fmat = f_ref[...]
    emat = e_ref[...]

    h_all = jnp.maximum(
        jax.lax.dot_general(xb, w1c_ref[...], (((1,), (0,)), ((), ())),
                            preferred_element_type=jnp.float32), 0.0)
    eo_parts = [
        jax.lax.dot_general(h_all[:, p * DH:(p + 1) * DH], w2_ref[p],
                            (((1,), (0,)), ((), ())),
                            preferred_element_type=jnp.float32)
        for p in range(P)
    ]
    eo_all = jnp.concatenate(eo_parts, axis=1)
    norm = jnp.sqrt(jax.lax.dot_general(
        eo_all * eo_all, fmat, (((1,), (0,)), ((), ())),
        preferred_element_type=jnp.float32))

    def one(p_ref, g_ref, syn_ref):
        raw = jax.lax.dot_general(xb, p_ref[...], (((1,), (1,)), ((), ())),
                                  preferred_element_type=jnp.float32)
        raw = raw * _INV_SQRT_DH - g_ref[...]
        logit = jnp.maximum(raw, 0.0)
        w = jnp.where(logit > 1e-6, logit, 0.0)
        wrep = jax.lax.dot_general(w, fmat, (((1,), (1,)), ((), ())),
                                   preferred_element_type=jnp.float32)
        syn_ref[...] = jax.lax.dot_general(
            eo_all * wrep, emat, (((1,), (0,)), ((), ())),
            preferred_element_type=jnp.float32)
        return logit, w * norm

    logq, rawq = one(pq_ref, gq_ref, synq_ref)
    logk, rawk = one(pk_ref, gk_ref, synk_ref)
    logv, rawv = one(pv_ref, gv_ref, synv_ref)
    sml_ref[0] = logq
    sml_ref[1] = logk
    sml_ref[2] = logv
    sml_ref[3] = rawq
    sml_ref[4] = rawk
    sml_ref[5] = rawv


def _compose(xf, w1cat, W2, fmat, emat,
             proto_q, gate_q, proto_k, gate_k, proto_v, gate_v, bs=512):
    grid = (S // bs,)
    row = pl.BlockSpec((bs, DH), lambda i: (i, 0))
    small = pl.BlockSpec((6, bs, P), lambda i: (0, i, 0))
    full = lambda shape: pl.BlockSpec(shape, lambda i: tuple(0 for _ in shape))
    out_shapes = (
        [jax.ShapeDtypeStruct((S, DH), jnp.float32)] * 3
        + [jax.ShapeDtypeStruct((6, S, P), jnp.float32)]
    )
    return pl.pallas_call(
        _compose_body,
        grid=grid,
        in_specs=[row, full((DH, P * DH)), full((P, DH, DH)),
                  full((P * DH, P)), full((P * DH, DH)),
                  full((P, DH)), full((1, P)),
                  full((P, DH)), full((1, P)),
                  full((P, DH)), full((1, P))],
        out_specs=[row, row, row, small],
        out_shape=out_shapes,
    )(xf, w1cat, W2, fmat, emat,
      proto_q, gate_q, proto_k, gate_k, proto_v, gate_v)


def _attn_body(q_ref, k_ref, v_ref, o_ref, *, bq, bk):
    i = pl.program_id(1)
    q = q_ref[0]

    def body(j, carry):
        acc, m, l = carry
        kb = k_ref[0, pl.ds(j * bk, bk), :]
        vb = v_ref[0, pl.ds(j * bk, bk), :]
        s = jax.lax.dot_general(q, kb, (((1,), (1,)), ((), ())),
                                preferred_element_type=jnp.float32)
        s = s * _INV_SQRT_DH
        m_new = jnp.maximum(m, jnp.max(s, axis=1, keepdims=True))
        alpha = jnp.exp(m - m_new)
        pmat = jnp.exp(s - m_new)
        l = l * alpha + jnp.sum(pmat, axis=1, keepdims=True)
        acc = acc * alpha + jax.lax.dot_general(
            pmat, vb, (((1,), (0,)), ((), ())),
            preferred_element_type=jnp.float32)
        return acc, m_new, l

    nfull = (i * bq) // bk
    acc = jnp.zeros((bq, DH), jnp.float32)
    m0 = jnp.full((bq, 1), -jnp.inf, jnp.float32)
    l0 = jnp.zeros((bq, 1), jnp.float32)
    acc, m, l = jax.lax.fori_loop(0, nfull, body, (acc, m0, l0))

    kb = k_ref[0, pl.ds(i * bq, bq), :]
    vb = v_ref[0, pl.ds(i * bq, bq), :]
    s = jax.lax.dot_general(q, kb, (((1,), (1,)), ((), ())),
                            preferred_element_type=jnp.float32)
    s = s * _INV_SQRT_DH
    rows = jax.lax.broadcasted_iota(jnp.int32, (bq, bq), 0)
    cols = jax.lax.broadcasted_iota(jnp.int32, (bq, bq), 1)
    s = jnp.where(rows >= cols, s, -jnp.inf)
    m_new = jnp.maximum(m, jnp.max(s, axis=1, keepdims=True))
    alpha = jnp.exp(m - m_new)
    pmat = jnp.exp(s - m_new)
    l = l * alpha + jnp.sum(pmat, axis=1, keepdims=True)
    acc = acc * alpha + jax.lax.dot_general(
        pmat, vb, (((1,), (0,)), ((), ())),
        preferred_element_type=jnp.float32)
    o_ref[0] = acc / l


def _attention(q, k, v, bq=512, bk=512):
    grid = (H, T // bq)
    qspec = pl.BlockSpec((1, bq, DH), lambda h, i: (h, i, 0))
    kvspec = pl.BlockSpec((1, T, DH), lambda h, i: (h, 0, 0))
    return pl.pallas_call(
        functools.partial(_attn_body, bq=bq, bk=bk),
        grid=grid,
        in_specs=[qspec, kvspec, kvspec],
        out_specs=qspec,
        out_shape=jax.ShapeDtypeStruct((H, T, DH), jnp.float32),
    )(q, k, v)


def _proj_body(x_ref, w_ref, o_ref):
    o_ref[...] = jax.lax.dot_general(x_ref[...], w_ref[...],
                                     (((1,), (1,)), ((), ())),
                                     preferred_element_type=jnp.float32)


def _out_proj(attn, Wo, br=512):
    grid = (T // br,)
    return pl.pallas_call(
        _proj_body,
        grid=grid,
        in_specs=[pl.BlockSpec((br, D_MODEL), lambda i: (i, 0)),
                  pl.BlockSpec((D_MODEL, D_MODEL), lambda i: (0, 0))],
        out_specs=pl.BlockSpec((br, D_MODEL), lambda i: (i, 0)),
        out_shape=jax.ShapeDtypeStruct((T, D_MODEL), jnp.float32),
    )(attn, Wo)


def kernel(x, position_ids, proto_q, gate_q, proto_k, gate_k, proto_v, gate_v,
           W1, W2, Wo):
    xf = x.reshape(S, DH)
    w1cat = jnp.transpose(W1, (1, 0, 2)).reshape(DH, P * DH)
    ridx = jnp.arange(P * DH, dtype=jnp.int32)
    fmat = (ridx[:, None] // DH == jnp.arange(P, dtype=jnp.int32)[None, :]
            ).astype(jnp.float32)
    emat = (ridx[:, None] % DH == jnp.arange(DH, dtype=jnp.int32)[None, :]
            ).astype(jnp.float32)

    (synq, synk, synv, sml) = _compose(
        xf, w1cat, W2, fmat, emat,
        proto_q, gate_q.reshape(1, P),
        proto_k, gate_k.reshape(1, P), proto_v, gate_v.reshape(1, P))

    q = synq.reshape(T, H, DH).transpose(1, 0, 2)
    k = synk.reshape(T, H, DH).transpose(1, 0, 2)
    v = synv.reshape(T, H, DH).transpose(1, 0, 2)
    attn = _attention(q, k, v).transpose(1, 0, 2).reshape(T, D_MODEL)
    out = _out_proj(attn, Wo).reshape(B, T, D_MODEL)

    shape_log = (B, T, H, P)
    return (out,
            sml[0].reshape(shape_log), sml[1].reshape(shape_log),
            sml[2].reshape(shape_log),
            sml[3], sml[4], sml[5])


# speedup vs baseline: 1.3997x; 1.1683x over previous
import math
import functools

import jax
import jax.numpy as jnp
from jax.experimental import pallas as pl

B = 1
T = 2048
D_MODEL = 768
H = 12
DH = D_MODEL // H
P = 8
S = B * T * H

_INV_SQRT_DH = 1.0 / math.sqrt(DH)


def _compose_body(x_ref, w1c_ref, w2_ref, f_ref, e_ref,
                  pq_ref, gq_ref, pk_ref, gk_ref, pv_ref, gv_ref,
                  synq_ref, synk_ref, synv_ref,
                  logq_ref, logk_ref, logv_ref,
                  rawq_ref, rawk_ref, rawv_ref):
    xb = x_ref[...]
    fmat = f_ref[...]
    emat = e_ref[...]

    h_all = jnp.maximum(
        jax.lax.dot_general(xb, w1c_ref[...], (((1,), (0,)), ((), ())),
                            preferred_element_type=jnp.float32), 0.0)
    eo_parts = [
        jax.lax.dot_general(h_all[:, p * DH:(p + 1) * DH], w2_ref[p],
                            (((1,), (0,)), ((), ())),
                            preferred_element_type=jnp.float32)
        for p in range(P)
    ]
    eo_all = jnp.concatenate(eo_parts, axis=1)
    normT = jnp.sqrt(jax.lax.dot_general(
        fmat, eo_all * eo_all, (((0,), (1,)), ((), ())),
        preferred_element_type=jnp.float32))

    def one(p_ref, g_ref, syn_ref, log_ref, raw_ref):
        rawT = jax.lax.dot_general(p_ref[...], xb, (((1,), (1,)), ((), ())),
                                   preferred_element_type=jnp.float32)
        rawT = rawT * _INV_SQRT_DH - g_ref[...]
        logitT = jnp.maximum(rawT, 0.0)
        wT = jnp.where(logitT > 1e-6, logitT, 0.0)
        wrep = jax.lax.dot_general(wT, fmat, (((0,), (1,)), ((), ())),
                                   preferred_element_type=jnp.float32)
        syn_ref[...] = jax.lax.dot_general(
            eo_all * wrep, emat, (((1,), (0,)), ((), ())),
            preferred_element_type=jnp.float32)
        log_ref[...] = logitT
        raw_ref[...] = wT * normT

    one(pq_ref, gq_ref, synq_ref, logq_ref, rawq_ref)
    one(pk_ref, gk_ref, synk_ref, logk_ref, rawk_ref)
    one(pv_ref, gv_ref, synv_ref, logv_ref, rawv_ref)


def _compose(xf, w1cat, W2, fmat, emat,
             proto_q, gate_q, proto_k, gate_k, proto_v, gate_v, bs=512):
    grid = (S // bs,)
    row = pl.BlockSpec((bs, DH), lambda i: (i, 0))
    small = pl.BlockSpec((P, bs), lambda i: (0, i))
    full = lambda shape: pl.BlockSpec(shape, lambda i: tuple(0 for _ in shape))
    out_shapes = (
        [jax.ShapeDtypeStruct((S, DH), jnp.float32)] * 3
        + [jax.ShapeDtypeStruct((P, S), jnp.float32)] * 6
    )
    return pl.pallas_call(
        _compose_body,
        grid=grid,
        in_specs=[row, full((DH, P * DH)), full((P, DH, DH)),
                  full((P * DH, P)), full((P * DH, DH)),
                  full((P, DH)), full((P, 1)),
                  full((P, DH)), full((P, 1)),
                  full((P, DH)), full((P, 1))],
        out_specs=[row, row, row, small, small, small, small, small, small],
        out_shape=out_shapes,
    )(xf, w1cat, W2, fmat, emat,
      proto_q, gate_q, proto_k, gate_k, proto_v, gate_v)


def _attn_body(q_ref, k_ref, v_ref, o_ref, *, bq, bk):
    i = pl.program_id(1)
    q = q_ref[0]

    def body(j, carry):
        acc, m, l = carry
        kb = k_ref[0, pl.ds(j * bk, bk), :]
        vb = v_ref[0, pl.ds(j * bk, bk), :]
        s = jax.lax.dot_general(q, kb, (((1,), (1,)), ((), ())),
                                preferred_element_type=jnp.float32)
        s = s * _INV_SQRT_DH
        m_new = jnp.maximum(m, jnp.max(s, axis=1, keepdims=True))
        alpha = jnp.exp(m - m_new)
        pmat = jnp.exp(s - m_new)
        l = l * alpha + jnp.sum(pmat, axis=1, keepdims=True)
        acc = acc * alpha + jax.lax.dot_general(
            pmat, vb, (((1,), (0,)), ((), ())),
            preferred_element_type=jnp.float32)
        return acc, m_new, l

    nfull = (i * bq) // bk
    acc = jnp.zeros((bq, DH), jnp.float32)
    m0 = jnp.full((bq, 1), -jnp.inf, jnp.float32)
    l0 = jnp.zeros((bq, 1), jnp.float32)
    acc, m, l = jax.lax.fori_loop(0, nfull, body, (acc, m0, l0))

    kb = k_ref[0, pl.ds(i * bq, bq), :]
    vb = v_ref[0, pl.ds(i * bq, bq), :]
    s = jax.lax.dot_general(q, kb, (((1,), (1,)), ((), ())),
                            preferred_element_type=jnp.float32)
    s = s * _INV_SQRT_DH
    rows = jax.lax.broadcasted_iota(jnp.int32, (bq, bq), 0)
    cols = jax.lax.broadcasted_iota(jnp.int32, (bq, bq), 1)
    s = jnp.where(rows >= cols, s, -jnp.inf)
    m_new = jnp.maximum(m, jnp.max(s, axis=1, keepdims=True))
    alpha = jnp.exp(m - m_new)
    pmat = jnp.exp(s - m_new)
    l = l * alpha + jnp.sum(pmat, axis=1, keepdims=True)
    acc = acc * alpha + jax.lax.dot_general(
        pmat, vb, (((1,), (0,)), ((), ())),
        preferred_element_type=jnp.float32)
    o_ref[0] = acc / l


def _attention(q, k, v, bq=512, bk=512):
    grid = (H, T // bq)
    qspec = pl.BlockSpec((1, bq, DH), lambda h, i: (h, i, 0))
    kvspec = pl.BlockSpec((1, T, DH), lambda h, i: (h, 0, 0))
    return pl.pallas_call(
        functools.partial(_attn_body, bq=bq, bk=bk),
        grid=grid,
        in_specs=[qspec, kvspec, kvspec],
        out_specs=qspec,
        out_shape=jax.ShapeDtypeStruct((H, T, DH), jnp.float32),
    )(q, k, v)


def _proj_body(x_ref, w_ref, o_ref):
    o_ref[...] = jax.lax.dot_general(x_ref[...], w_ref[...],
                                     (((1,), (1,)), ((), ())),
                                     preferred_element_type=jnp.float32)


def _out_proj(attn, Wo, br=512):
    grid = (T // br,)
    return pl.pallas_call(
        _proj_body,
        grid=grid,
        in_specs=[pl.BlockSpec((br, D_MODEL), lambda i: (i, 0)),
                  pl.BlockSpec((D_MODEL, D_MODEL), lambda i: (0, 0))],
        out_specs=pl.BlockSpec((br, D_MODEL), lambda i: (i, 0)),
        out_shape=jax.ShapeDtypeStruct((T, D_MODEL), jnp.float32),
    )(attn, Wo)


def kernel(x, position_ids, proto_q, gate_q, proto_k, gate_k, proto_v, gate_v,
           W1, W2, Wo):
    xf = x.reshape(S, DH)
    w1cat = jnp.transpose(W1, (1, 0, 2)).reshape(DH, P * DH)
    ridx = jnp.arange(P * DH, dtype=jnp.int32)
    fmat = (ridx[:, None] // DH == jnp.arange(P, dtype=jnp.int32)[None, :]
            ).astype(jnp.float32)
    emat = (ridx[:, None] % DH == jnp.arange(DH, dtype=jnp.int32)[None, :]
            ).astype(jnp.float32)

    (synq, synk, synv, logq, logk, logv, rawq, rawk, rawv) = _compose(
        xf, w1cat, W2, fmat, emat,
        proto_q, gate_q.reshape(P, 1),
        proto_k, gate_k.reshape(P, 1), proto_v, gate_v.reshape(P, 1))

    q = synq.reshape(T, H, DH).transpose(1, 0, 2)
    k = synk.reshape(T, H, DH).transpose(1, 0, 2)
    v = synv.reshape(T, H, DH).transpose(1, 0, 2)
    attn = _attention(q, k, v).transpose(1, 0, 2).reshape(T, D_MODEL)
    out = _out_proj(attn, Wo).reshape(B, T, D_MODEL)

    shape_log = (B, T, H, P)
    return (out,
            logq.T.reshape(shape_log), logk.T.reshape(shape_log),
            logv.T.reshape(shape_log),
            rawq.T, rawk.T, rawv.T)
